# Initial kernel scaffold; baseline (speedup 1.0000x reference)
import jax
import jax.numpy as jnp
from jax.experimental import pallas as pl


def kernel(z_e_x, codebook):
    raise NotImplementedError("write your pallas kernel here")


# speedup vs baseline: 1.3104x; 1.3104x over previous
import functools

import jax
import jax.numpy as jnp
from jax import lax
from jax.experimental import pallas as pl
from jax.experimental.pallas import tpu as pltpu
from jax.experimental.pallas import tpu_sc as plsc

_B, _S, _D, _K = 16, 1024, 256, 8192
_N = _B * _S
_T = 256
_NB = _N // _T


def _argmin_body(csq_ref, isq_ref, x_ref, cb_ref, out_ref):
    mm = lax.dot_general(
        x_ref[...], cb_ref[...],
        dimension_numbers=(((1,), (1,)), ((), ())),
        preferred_element_type=jnp.float32,
    )
    d = (csq_ref[...] + isq_ref[...]) - 2.0 * mm
    m = jnp.min(d, axis=1, keepdims=True)
    ii = lax.broadcasted_iota(jnp.int32, d.shape, 1)
    out_ref[0, 0, :] = jnp.min(jnp.where(d == m, ii, _K), axis=1)


def _compute_indices(flat, codebook, csq, isq):
    return pl.pallas_call(
        _argmin_body,
        grid=(_NB,),
        in_specs=[
            pl.BlockSpec((1, _K), lambda i: (0, 0)),
            pl.BlockSpec((_T, 1), lambda i: (i, 0)),
            pl.BlockSpec((_T, _D), lambda i: (i, 0)),
            pl.BlockSpec((_K, _D), lambda i: (0, 0)),
        ],
        out_specs=pl.BlockSpec((1, 1, _T), lambda i: (i, 0, 0)),
        out_shape=jax.ShapeDtypeStruct((_NB, 1, _T), jnp.int32),
    )(csq.reshape(1, _K), isq, flat, codebook)


_SC_CHUNK = 128


def _sc_gather(codebook, idx_flat):
    info = plsc.get_sparse_core_info()
    num_workers = info.num_cores * info.num_subcores
    b_per_w = _N // num_workers
    mesh = plsc.VectorSubcoreMesh(core_axis_name="c", subcore_axis_name="s")

    @functools.partial(
        pl.kernel, mesh=mesh,
        out_type=jax.ShapeDtypeStruct((_N, _D), jnp.float32),
        scratch_types=[
            pltpu.VMEM((b_per_w,), jnp.int32),
            pltpu.VMEM((_SC_CHUNK, _D), jnp.float32),
            pltpu.SemaphoreType.DMA,
        ],
    )
    def k(table_hbm, idx_hbm, out_hbm, idx_v, rows_v, sem):
        wid = lax.axis_index("s") * info.num_cores + lax.axis_index("c")
        base = wid * b_per_w
        pltpu.sync_copy(idx_hbm.at[pl.ds(base, b_per_w)], idx_v)

        @pl.loop(0, b_per_w // _SC_CHUNK)
        def _(j):
            idx_chunk = idx_v.at[pl.ds(j * _SC_CHUNK, _SC_CHUNK)]
            pltpu.async_copy(table_hbm.at[idx_chunk], rows_v, sem).wait()
            pltpu.sync_copy(rows_v, out_hbm.at[pl.ds(base + j * _SC_CHUNK, _SC_CHUNK)])

    return k(codebook, idx_flat)


def kernel(z_e_x, codebook):
    flat = z_e_x.reshape(-1, _D)
    csq = jnp.sum(codebook ** 2, axis=1)
    isq = jnp.sum(flat ** 2, axis=1, keepdims=True)
    idx_flat = _compute_indices(flat, codebook, csq, isq).reshape(-1)
    codes = _sc_gather(codebook, idx_flat)
    z_q = codes.reshape(z_e_x.shape)
    return (z_q, z_q, idx_flat.reshape(_B, _S))
